# baseline (speedup 1.0000x reference)
import functools

import jax
import jax.numpy as jnp
from jax.experimental import pallas as pl
from jax.experimental.pallas import tpu as pltpu


def _dot_t(a, w):
    return jax.lax.dot_general(a, w, (((1,), (1,)), ((), ())),
                               preferred_element_type=jnp.float32)


def _mlp_kernel(x_ref, w1_ref, b1_ref, w2_ref, b2_ref, w3_ref, b3_ref, o_hbm,
                o_buf, out_sem, *, tb, n_steps):
    i = pl.program_id(0)
    rows = tb // 8
    o3 = o_hbm.reshape(o_hbm.shape[0] // 8, 8, o_hbm.shape[1])

    def out_copy(slot, step):
        return pltpu.make_async_copy(
            o_buf.at[slot], o3.at[pl.ds(step * rows, rows)], out_sem.at[slot])

    cur = jax.lax.rem(i, 2)

    @pl.when(i >= 2)
    def _():
        out_copy(cur, 0).wait()

    h1 = jnp.maximum(_dot_t(x_ref[...], w1_ref[...]) + b1_ref[...].T, 0.0)
    h2 = jnp.maximum(_dot_t(h1, w2_ref[...]) + b2_ref[...].T, 0.0)
    o = _dot_t(h2, w3_ref[...]) + b3_ref[...].T
    o_buf[cur] = o.reshape(rows, 8, o_hbm.shape[1])

    out_copy(cur, i).start()

    @pl.when(i == n_steps - 1)
    def _():
        @pl.when(n_steps >= 2)
        def _():
            out_copy(jax.lax.rem(i + 1, 2), 0).wait()
        out_copy(cur, 0).wait()


def kernel(x, w1, b1, w2, b2, w3, b3):
    B, F = x.shape
    H1, H2, O = w1.shape[0], w2.shape[0], w3.shape[0]

    TB = min(B, 16384)
    Bp = pl.cdiv(B, TB) * TB
    if Bp != B:
        x = jnp.pad(x, ((0, Bp - B), (0, 0)))
    n_steps = Bp // TB

    out = pl.pallas_call(
        functools.partial(_mlp_kernel, tb=TB, n_steps=n_steps),
        out_shape=jax.ShapeDtypeStruct((Bp, O), jnp.float32),
        grid=(n_steps,),
        in_specs=[
            pl.BlockSpec((TB, F), lambda i: (i, 0)),
            pl.BlockSpec((H1, F), lambda i: (0, 0)),
            pl.BlockSpec((H1, 1), lambda i: (0, 0)),
            pl.BlockSpec((H2, H1), lambda i: (0, 0)),
            pl.BlockSpec((H2, 1), lambda i: (0, 0)),
            pl.BlockSpec((O, H2), lambda i: (0, 0)),
            pl.BlockSpec((O, 1), lambda i: (0, 0)),
        ],
        out_specs=pl.BlockSpec(memory_space=pl.ANY),
        scratch_shapes=[
            pltpu.VMEM((2, TB // 8, 8, O), jnp.float32),
            pltpu.SemaphoreType.DMA((2,)),
        ],
        compiler_params=pltpu.CompilerParams(
            dimension_semantics=("arbitrary",),
            vmem_limit_bytes=64 * 1024 * 1024,
        ),
        cost_estimate=pl.CostEstimate(
            flops=2 * B * (F * H1 + H1 * H2 + H2 * O),
            transcendentals=0,
            bytes_accessed=4 * (B * F + B * O + F * H1 + H1 + H1 * H2 + H2 + H2 * O + O),
        ),
    )(x, w1, b1, w2, b2, w3, b3)

    return out[:B]


# speedup vs baseline: 1.0010x; 1.0010x over previous
import functools

import jax
import jax.numpy as jnp
from jax.experimental import pallas as pl
from jax.experimental.pallas import tpu as pltpu


def _dot_t(a, w):
    return jax.lax.dot_general(a, w, (((1,), (1,)), ((), ())),
                               preferred_element_type=jnp.float32)


def _mlp_kernel(x_ref, w1_ref, b1_ref, w2_ref, b2_ref, w3_ref, b3_ref, o_hbm,
                o_buf, out_sem, *, tb, n_steps):
    i = pl.program_id(0)
    rows = tb // 8
    o3 = o_hbm.reshape(o_hbm.shape[0] // 8, 8, o_hbm.shape[1])

    def out_copy(slot, step):
        return pltpu.make_async_copy(
            o_buf.at[slot], o3.at[pl.ds(step * rows, rows)], out_sem.at[slot])

    cur = jax.lax.rem(i, 2)

    @pl.when(i >= 2)
    def _():
        out_copy(cur, 0).wait()

    h1 = jnp.maximum(_dot_t(x_ref[...], w1_ref[...]) + b1_ref[...].T, 0.0)
    h2 = jnp.maximum(_dot_t(h1, w2_ref[...]) + b2_ref[...].T, 0.0)
    o = _dot_t(h2, w3_ref[...]) + b3_ref[...].T
    o_buf[cur] = o.reshape(rows, 8, o_hbm.shape[1])

    out_copy(cur, i).start()

    @pl.when(i == n_steps - 1)
    def _():
        @pl.when(n_steps >= 2)
        def _():
            out_copy(jax.lax.rem(i + 1, 2), 0).wait()
        out_copy(cur, 0).wait()


def kernel(x, w1, b1, w2, b2, w3, b3):
    B, F = x.shape
    H1, H2, O = w1.shape[0], w2.shape[0], w3.shape[0]

    TB = min(B, 16384)
    Bp = pl.cdiv(B, TB) * TB
    if Bp != B:
        x = jnp.pad(x, ((0, Bp - B), (0, 0)))
    n_steps = Bp // TB

    out = pl.pallas_call(
        functools.partial(_mlp_kernel, tb=TB, n_steps=n_steps),
        out_shape=jax.ShapeDtypeStruct((Bp, O), jnp.float32),
        grid=(n_steps,),
        in_specs=[
            pl.BlockSpec((TB, F), lambda i: (i, 0)),
            pl.BlockSpec((H1, F), lambda i: (0, 0)),
            pl.BlockSpec((H1, 1), lambda i: (0, 0)),
            pl.BlockSpec((H2, H1), lambda i: (0, 0)),
            pl.BlockSpec((H2, 1), lambda i: (0, 0)),
            pl.BlockSpec((O, H2), lambda i: (0, 0)),
            pl.BlockSpec((O, 1), lambda i: (0, 0)),
        ],
        out_specs=pl.BlockSpec(memory_space=pl.ANY),
        scratch_shapes=[
            pltpu.VMEM((2, TB // 8, 8, O), jnp.float32),
            pltpu.SemaphoreType.DMA((2,)),
        ],
        compiler_params=pltpu.CompilerParams(
            dimension_semantics=("arbitrary",),
            vmem_limit_bytes=64 * 1024 * 1024,
        ),
        cost_estimate=pl.CostEstimate(
            flops=2 * B * (F * H1 + H1 * H2 + H2 * O),
            transcendentals=0,
            bytes_accessed=4 * (B * F + B * O + F * H1 + H1 + H1 * H2 + H2 + H2 * O + O),
        ),
    )(x, w1, b1, w2, b2, w3, b3)

    return out if Bp == B else out[:B]
